# Initial kernel scaffold; baseline (speedup 1.0000x reference)
import jax
import jax.numpy as jnp
from jax.experimental import pallas as pl


def kernel(x, edge_index, W1, b1, W2, b2):
    raise NotImplementedError("write your pallas kernel here")


# speedup vs baseline: 36.2916x; 36.2916x over previous
import jax
import jax.numpy as jnp
from jax.experimental import pallas as pl

_N = 100
_N_PAD = 128
_E = 6400


def _sg_kernel(x_ref, src_ref, dst_ref, w1_ref, b1_ref, w2_ref, b2_ref, o_ref):
    f32 = jnp.float32
    iota_ne = jax.lax.broadcasted_iota(jnp.int32, (_N_PAD, _E), 0)
    st = (src_ref[:, :] == iota_ne).astype(f32)
    dt = (dst_ref[:, :] == iota_ne).astype(f32)
    A = jax.lax.dot_general(dt, st, (((1,), (1,)), ((), ())),
                            preferred_element_type=f32)
    row = jax.lax.broadcasted_iota(jnp.int32, (_N_PAD, _N_PAD), 0)
    col = jax.lax.broadcasted_iota(jnp.int32, (_N_PAD, _N_PAD), 1)
    A = A + jnp.where((row == col) & (row < _N), 1.0, 0.0).astype(f32)
    deg = jnp.sum(A, axis=1, keepdims=True)
    dis = jnp.where(deg > 0.0, jax.lax.rsqrt(deg), 0.0)
    z1 = dis * jnp.dot(x_ref[:, :], w1_ref[:, :], preferred_element_type=f32)
    h = jnp.maximum(
        dis * jnp.dot(A, z1, preferred_element_type=f32) + b1_ref[:, :], 0.0)
    z2 = dis * jnp.dot(h, w2_ref[:, :], preferred_element_type=f32)
    o_ref[:, :] = dis * jnp.dot(A, z2, preferred_element_type=f32) + b2_ref[:, :]


def kernel(x, edge_index, W1, b1, W2, b2):
    xpad = jnp.zeros((_N_PAD, x.shape[1]), x.dtype).at[: x.shape[0]].set(x)
    src = edge_index[0].astype(jnp.int32).reshape(1, _E)
    dst = edge_index[1].astype(jnp.int32).reshape(1, _E)
    out = pl.pallas_call(
        _sg_kernel,
        out_shape=jax.ShapeDtypeStruct((_N_PAD, W2.shape[0]), jnp.float32),
    )(xpad, src, dst, W1.T, b1.reshape(1, -1), W2.T, b2.reshape(1, -1))
    return out[:_N].reshape(_N * W2.shape[0])


# speedup vs baseline: 86.6478x; 2.3875x over previous
import jax
import jax.numpy as jnp
from jax.experimental import pallas as pl

_N = 100
_E = 6400

_NT = (((1,), (1,)), ((), ()))


def _sg_kernel(x_ref, ei_ref, w1_ref, b1_ref, w2_ref, b2_ref, o_ref):
    f32 = jnp.float32
    src = ei_ref[0:1, :]
    dst = ei_ref[1:2, :]
    iota_ne = jax.lax.broadcasted_iota(jnp.int32, (_N, _E), 0)
    st = (src == iota_ne).astype(f32)
    dt = (dst == iota_ne).astype(f32)
    A = jax.lax.dot_general(dt, st, _NT, preferred_element_type=f32)
    row = jax.lax.broadcasted_iota(jnp.int32, (_N, _N), 0)
    col = jax.lax.broadcasted_iota(jnp.int32, (_N, _N), 1)
    A = A + jnp.where(row == col, 1.0, 0.0).astype(f32)
    deg = jnp.sum(A, axis=1, keepdims=True)
    dis = jax.lax.rsqrt(deg)
    z1 = dis * jax.lax.dot_general(x_ref[:, :], w1_ref[:, :], _NT,
                                   preferred_element_type=f32)
    h = jnp.maximum(
        dis * jnp.dot(A, z1, preferred_element_type=f32) + b1_ref[:, :], 0.0)
    z2 = dis * jax.lax.dot_general(h, w2_ref[:, :], _NT,
                                   preferred_element_type=f32)
    o_ref[:, :] = dis * jnp.dot(A, z2, preferred_element_type=f32) + b2_ref[:, :]


def kernel(x, edge_index, W1, b1, W2, b2):
    out = pl.pallas_call(
        _sg_kernel,
        out_shape=jax.ShapeDtypeStruct((_N, W2.shape[0]), jnp.float32),
    )(x, edge_index.astype(jnp.int32), W1, b1.reshape(1, -1),
      W2, b2.reshape(1, -1))
    return out.reshape(_N * W2.shape[0])
